# baseline (speedup 1.0000x reference)
import functools

import jax
import jax.numpy as jnp
from jax import lax
from jax.experimental import pallas as pl
from jax.experimental.pallas import tpu as pltpu
from jax.experimental.pallas import tpu_sc as plsc

DIM = 16


def _dyn_gather(v, idx):
    dnums = lax.GatherDimensionNumbers(
        offset_dims=(), collapsed_slice_dims=(0,), start_index_map=(0,)
    )
    return lax.gather(
        v,
        idx[:, None],
        dnums,
        slice_sizes=(1,),
        mode=lax.GatherScatterMode.PROMISE_IN_BOUNDS,
    )


_info = plsc.get_sparse_core_info()
_NC, _NS, _NL = _info.num_cores, _info.num_subcores, _info.num_lanes
_NW = _NC * _NS


@functools.lru_cache(maxsize=None)
def _build(n_idx: int, vocab: int, chunk: int):
    per_w = n_idx // _NW
    n_chunks = per_w // chunk
    n_fire = chunk // 128
    n_groups = chunk // _NL

    mesh = plsc.VectorSubcoreMesh(core_axis_name="c", subcore_axis_name="s")

    @functools.partial(
        pl.kernel,
        mesh=mesh,
        compiler_params=pltpu.CompilerParams(use_tc_tiling_on_sc=False),
        out_type=jax.ShapeDtypeStruct((n_idx, DIM), jnp.float32),
        scratch_types=[
            pltpu.VMEM((chunk,), jnp.int32),
            pltpu.VMEM((chunk, DIM), jnp.float32),
            pltpu.SemaphoreType.DMA,
        ],
    )
    def k(x_hbm, w_hbm, out_hbm, idx_v, rows_v, sem):
        wid = lax.axis_index("s") * _NC + lax.axis_index("c")
        iota = lax.iota(jnp.int32, _NL)

        def chunk_body(ci, carry):
            base = wid * per_w + ci * chunk
            pltpu.sync_copy(x_hbm.at[pl.ds(base, chunk)], idx_v)
            cps = [
                pltpu.async_copy(
                    w_hbm.at[idx_v.at[pl.ds(j * 128, 128)]],
                    rows_v.at[pl.ds(j * 128, 128)],
                    sem,
                )
                for j in range(n_fire)
            ]
            for cp in cps:
                cp.wait()

            def group_body(g, carry2):
                b0 = g * _NL
                idx16 = idx_v[pl.ds(b0, _NL)]
                for r in range(_NL):
                    row = rows_v[b0 + r]
                    e = jnp.exp(row)
                    s = e
                    for sh in (1, 2, 4, 8):
                        s = s + _dyn_gather(s, iota ^ sh)
                    flag = jnp.where(idx16[r] == 0, jnp.float32(0.0),
                                     jnp.float32(1.0))
                    rows_v[b0 + r] = e * (flag / s)
                return carry2

            lax.fori_loop(0, n_groups, group_body, 0)
            pltpu.sync_copy(rows_v, out_hbm.at[pl.ds(base, chunk)])
            return carry

        lax.fori_loop(0, n_chunks, chunk_body, 0)

    return k


def kernel(x, weight):
    b, h = x.shape
    vocab, dim = weight.shape
    n_idx = b * h
    out = _build(n_idx, vocab, 1280)(x.reshape(n_idx), weight)
    return out.reshape(b, h, dim)


# speedup vs baseline: 1.1238x; 1.1238x over previous
import functools

import jax
import jax.numpy as jnp
from jax import lax
from jax.experimental import pallas as pl
from jax.experimental.pallas import tpu as pltpu
from jax.experimental.pallas import tpu_sc as plsc

DIM = 16


def _dyn_gather(v, idx):
    dnums = lax.GatherDimensionNumbers(
        offset_dims=(), collapsed_slice_dims=(0,), start_index_map=(0,)
    )
    return lax.gather(
        v,
        idx[:, None],
        dnums,
        slice_sizes=(1,),
        mode=lax.GatherScatterMode.PROMISE_IN_BOUNDS,
    )


_info = plsc.get_sparse_core_info()
_NC, _NS, _NL = _info.num_cores, _info.num_subcores, _info.num_lanes
_NW = _NC * _NS


@functools.lru_cache(maxsize=None)
def _build(batch: int, hist: int, vocab: int, rows_per_chunk: int):
    n_idx = batch * hist
    per_w = n_idx // _NW
    chunk = rows_per_chunk * hist
    n_chunks = per_w // chunk
    n_fire = chunk // 128
    n_groups = chunk // _NL

    mesh = plsc.VectorSubcoreMesh(core_axis_name="c", subcore_axis_name="s")

    @functools.partial(
        pl.kernel,
        mesh=mesh,
        compiler_params=pltpu.CompilerParams(use_tc_tiling_on_sc=False),
        out_type=jax.ShapeDtypeStruct((batch, hist, DIM), jnp.float32),
        scratch_types=[
            pltpu.VMEM((chunk,), jnp.int32),
            pltpu.VMEM((chunk, DIM), jnp.float32),
            pltpu.VMEM((rows_per_chunk, hist, DIM), jnp.float32),
            pltpu.SemaphoreType.DMA,
        ],
    )
    def k(x_hbm, w_hbm, out_hbm, idx_v, rows_v, out_v, sem):
        wid = lax.axis_index("s") * _NC + lax.axis_index("c")
        iota = lax.iota(jnp.int32, _NL)

        def chunk_body(ci, carry):
            base = wid * per_w + ci * chunk
            pltpu.sync_copy(x_hbm.at[pl.ds(base, chunk)], idx_v)
            cps = [
                pltpu.async_copy(
                    w_hbm.at[idx_v.at[pl.ds(j * 128, 128)]],
                    rows_v.at[pl.ds(j * 128, 128)],
                    sem,
                )
                for j in range(n_fire)
            ]
            for cp in cps:
                cp.wait()

            def group_body(g, carry2):
                b0 = g * _NL
                idx16 = idx_v[pl.ds(b0, _NL)]
                for r in range(_NL):
                    row = rows_v[b0 + r]
                    e = jnp.exp(row)
                    s = e
                    for sh in (1, 2, 4, 8):
                        s = s + _dyn_gather(s, iota ^ sh)
                    flag = jnp.where(idx16[r] == 0, jnp.float32(0.0),
                                     jnp.float32(1.0))
                    rows_v[b0 + r] = e * (flag / s)
                return carry2

            lax.fori_loop(0, n_groups, group_body, 0)

            def repack_body(i, carry3):
                def inner(j, carry4):
                    out_v[i, j] = rows_v[i * hist + j]
                    return carry4

                lax.fori_loop(0, hist, inner, 0)
                return carry3

            lax.fori_loop(0, rows_per_chunk, repack_body, 0)
            pltpu.sync_copy(
                out_v,
                out_hbm.at[pl.ds(wid * (per_w // hist) + ci * rows_per_chunk,
                                 rows_per_chunk)],
            )
            return carry

        lax.fori_loop(0, n_chunks, chunk_body, 0)

    return k


def kernel(x, weight):
    b, h = x.shape
    vocab, dim = weight.shape
    return _build(b, h, vocab, 64)(x.reshape(b * h), weight)
